# baseline (speedup 1.0000x reference)
import functools

import jax
import jax.numpy as jnp
from jax.experimental import pallas as pl
from jax.experimental.pallas import tpu as pltpu


TM = 128
TF = 512


def _fused_moe_body(tid_ref, gid_ref, rlo_ref, rhi_ref,
                    x_ref, w_ref, wg_ref, wu_ref, wd_ref, out_ref):
    f = pl.program_id(0)
    t = pl.program_id(1)

    base = tid_ref[t] * TM
    x = x_ref[pl.ds(base, TM), :]
    wg = wg_ref[0].astype(jnp.bfloat16)
    wu = wu_ref[0].astype(jnp.bfloat16)
    wd = wd_ref[0].astype(jnp.bfloat16)
    gate = jnp.dot(x, wg, preferred_element_type=jnp.float32)
    up = jnp.dot(x, wu, preferred_element_type=jnp.float32)
    fused = gate * jax.lax.logistic(gate) * up
    fused = fused * w_ref[pl.ds(base, TM), :]
    part = jnp.dot(fused.astype(jnp.bfloat16), wd, preferred_element_type=jnp.float32)

    row = base + jax.lax.broadcasted_iota(jnp.int32, (TM, 1), 0)
    mask = (row >= rlo_ref[t]) & (row < rhi_ref[t])
    part = jnp.where(mask, part, 0.0)

    prev_t = jnp.maximum(t - 1, 0)
    first_visit = (f == 0) & ((t == 0) | (tid_ref[t] != tid_ref[prev_t]))

    @pl.when(first_visit)
    def _():
        out_ref[pl.ds(base, TM), :] = part

    @pl.when(jnp.logical_not(first_visit))
    def _():
        out_ref[pl.ds(base, TM), :] += part


@functools.partial(jax.jit, static_argnums=())
def kernel(x_TD, router_weights_TX, selected_experts_TX,
           kernel_gating, kernel_up_proj, kernel_down_proj):
    T, D = x_TD.shape
    K = router_weights_TX.shape[1]
    E, _, F = kernel_gating.shape
    M = T * K
    m_tiles = M // TM
    NL = m_tiles + E - 1
    NF = F // TF

    flat = selected_experts_TX.reshape(-1)
    oh = (flat[:, None] == jnp.arange(E)[None, :]).astype(jnp.int32)
    csum = jnp.cumsum(oh, axis=0)
    sizes = csum[-1]
    ends = jnp.cumsum(sizes)
    starts = ends - sizes
    rank = jnp.sum(oh * csum, axis=1) - 1
    pos = jnp.sum(oh * starts[None, :], axis=1) + rank

    slot_iota = jnp.arange(M, dtype=jnp.int32)
    tok_sorted = jnp.zeros((M,), jnp.int32).at[pos].set(slot_iota // K)
    x_sorted = jnp.take(x_TD.astype(jnp.bfloat16), tok_sorted, axis=0)
    w_sorted = jnp.zeros((M,), jnp.float32).at[pos].set(
        router_weights_TX.reshape(-1))[:, None]

    nonempty = sizes > 0
    first_tile = jnp.where(nonempty, starts // TM, 0)
    last_tile = jnp.where(nonempty, (ends - 1) // TM, -1)
    ntiles = jnp.maximum(last_tile - first_tile + 1, 0)
    work_start = jnp.concatenate([jnp.zeros(1, ntiles.dtype), jnp.cumsum(ntiles)[:-1]])
    S = jnp.sum(ntiles)
    j = jnp.arange(NL)
    g_j = jnp.searchsorted(work_start, j, side='right') - 1
    valid = j < S
    tile_ids = jnp.where(valid, first_tile[g_j] + (j - work_start[g_j]),
                         m_tiles - 1).astype(jnp.int32)
    row_lo = jnp.where(valid, jnp.maximum(starts[g_j], tile_ids * TM), 0).astype(jnp.int32)
    row_hi = jnp.where(valid, jnp.minimum(ends[g_j], (tile_ids + 1) * TM), 0).astype(jnp.int32)
    group_ids = jnp.where(valid, g_j, E - 1).astype(jnp.int32)

    grid_spec = pltpu.PrefetchScalarGridSpec(
        num_scalar_prefetch=4,
        grid=(NF, NL),
        in_specs=[
            pl.BlockSpec((M, D), lambda f, t, tid, gid, rlo, rhi: (0, 0)),
            pl.BlockSpec((M, 1), lambda f, t, tid, gid, rlo, rhi: (0, 0)),
            pl.BlockSpec((1, D, TF), lambda f, t, tid, gid, rlo, rhi: (gid[t], 0, f)),
            pl.BlockSpec((1, D, TF), lambda f, t, tid, gid, rlo, rhi: (gid[t], 0, f)),
            pl.BlockSpec((1, TF, D), lambda f, t, tid, gid, rlo, rhi: (gid[t], f, 0)),
        ],
        out_specs=pl.BlockSpec((M, D), lambda f, t, tid, gid, rlo, rhi: (0, 0)),
    )
    y_sorted = pl.pallas_call(
        _fused_moe_body,
        grid_spec=grid_spec,
        out_shape=jax.ShapeDtypeStruct((M, D), jnp.float32),
    )(tile_ids, group_ids, row_lo, row_hi,
      x_sorted, w_sorted, kernel_gating, kernel_up_proj, kernel_down_proj)

    pos_TK = pos.reshape(T, K)
    out_TD = jnp.take(y_sorted, pos_TK[:, 0], axis=0)
    for k in range(1, K):
        out_TD = out_TD + jnp.take(y_sorted, pos_TK[:, k], axis=0)
    return out_TD.astype(jnp.float32)


# speedup vs baseline: 1.4773x; 1.4773x over previous
import functools

import jax
import jax.numpy as jnp
from jax.experimental import pallas as pl
from jax.experimental.pallas import tpu as pltpu


TM = 128
TF = 1024


def _fused_moe_body(ft_ref, nt_ref, st_ref, en_ref,
                    x_ref, w_ref, wg_ref, wu_ref, wd_ref, out_ref,
                    wg_bf, wu_bf, wd_bf):
    e = pl.program_id(0)
    f = pl.program_id(1)

    @pl.when((e == 0) & (f == 0))
    def _():
        out_ref[...] = jnp.zeros_like(out_ref)

    wg_bf[...] = wg_ref[0].astype(jnp.bfloat16)
    wu_bf[...] = wu_ref[0].astype(jnp.bfloat16)
    wd_bf[...] = wd_ref[0].astype(jnp.bfloat16)

    start = st_ref[e]
    end = en_ref[e]
    first = ft_ref[e]

    def chunk(c, carry):
        base = (first + c) * TM
        x = x_ref[pl.ds(base, TM), :]
        gate = jnp.dot(x, wg_bf[...], preferred_element_type=jnp.float32)
        up = jnp.dot(x, wu_bf[...], preferred_element_type=jnp.float32)
        fused = gate * jax.lax.logistic(gate) * up
        fused = fused * w_ref[pl.ds(base, TM), :]
        part = jnp.dot(fused.astype(jnp.bfloat16), wd_bf[...],
                       preferred_element_type=jnp.float32)
        row = base + jax.lax.broadcasted_iota(jnp.int32, (TM, 1), 0)
        mask = (row >= start) & (row < end)
        out_ref[pl.ds(base, TM), :] += jnp.where(mask, part, 0.0)
        return carry

    jax.lax.fori_loop(0, nt_ref[e], chunk, 0)


@functools.partial(jax.jit, static_argnums=())
def kernel(x_TD, router_weights_TX, selected_experts_TX,
           kernel_gating, kernel_up_proj, kernel_down_proj):
    T, D = x_TD.shape
    K = router_weights_TX.shape[1]
    E, _, F = kernel_gating.shape
    M = T * K
    m_tiles = M // TM
    NF = F // TF

    flat = selected_experts_TX.reshape(-1)
    oh = (flat[:, None] == jnp.arange(E)[None, :]).astype(jnp.int32)
    csum = jnp.cumsum(oh, axis=0)
    sizes = csum[-1]
    ends = jnp.cumsum(sizes)
    starts = ends - sizes
    rank = jnp.sum(oh * csum, axis=1) - 1
    pos = jnp.sum(oh * starts[None, :], axis=1) + rank

    slot_iota = jnp.arange(M, dtype=jnp.int32)
    tok_sorted = jnp.zeros((M,), jnp.int32).at[pos].set(slot_iota // K)
    x_sorted = jnp.take(x_TD.astype(jnp.bfloat16), tok_sorted, axis=0)
    w_sorted = jnp.zeros((M,), jnp.float32).at[pos].set(
        router_weights_TX.reshape(-1))[:, None]

    nonempty = sizes > 0
    first_tile = jnp.where(nonempty, starts // TM, 0).astype(jnp.int32)
    last_tile = jnp.where(nonempty, (ends - 1) // TM, -1)
    ntiles = jnp.maximum(last_tile - first_tile + 1, 0).astype(jnp.int32)

    grid_spec = pltpu.PrefetchScalarGridSpec(
        num_scalar_prefetch=4,
        grid=(E, NF),
        in_specs=[
            pl.BlockSpec((M, D), lambda e, f, ft, nt, st, en: (0, 0)),
            pl.BlockSpec((M, 1), lambda e, f, ft, nt, st, en: (0, 0)),
            pl.BlockSpec((1, D, TF), lambda e, f, ft, nt, st, en: (e, 0, f)),
            pl.BlockSpec((1, D, TF), lambda e, f, ft, nt, st, en: (e, 0, f)),
            pl.BlockSpec((1, TF, D), lambda e, f, ft, nt, st, en: (e, f, 0)),
        ],
        out_specs=pl.BlockSpec((M, D), lambda e, f, ft, nt, st, en: (0, 0)),
        scratch_shapes=[
            pltpu.VMEM((D, TF), jnp.bfloat16),
            pltpu.VMEM((D, TF), jnp.bfloat16),
            pltpu.VMEM((TF, D), jnp.bfloat16),
        ],
    )
    y_sorted = pl.pallas_call(
        _fused_moe_body,
        grid_spec=grid_spec,
        out_shape=jax.ShapeDtypeStruct((M, D), jnp.float32),
        compiler_params=pltpu.CompilerParams(vmem_limit_bytes=62 * 1024 * 1024),
    )(first_tile, ntiles, starts.astype(jnp.int32), ends.astype(jnp.int32),
      x_sorted, w_sorted, kernel_gating, kernel_up_proj, kernel_down_proj)

    pos_TK = pos.reshape(T, K)
    out_TD = jnp.take(y_sorted, pos_TK[:, 0], axis=0)
    for k in range(1, K):
        out_TD = out_TD + jnp.take(y_sorted, pos_TK[:, k], axis=0)
    return out_TD.astype(jnp.float32)


# speedup vs baseline: 1.5245x; 1.0320x over previous
import functools

import jax
import jax.numpy as jnp
from jax import lax
from jax.experimental import pallas as pl
from jax.experimental.pallas import tpu as pltpu
from jax.experimental.pallas import tpu_sc as plsc


TM = 128
TF = 1024


def _fused_moe_body(ft_ref, nt_ref, st_ref, en_ref,
                    x_ref, w_ref, wg_ref, wu_ref, wd_ref, out_ref,
                    wg_bf, wu_bf, wd_bf):
    e = pl.program_id(0)
    f = pl.program_id(1)

    @pl.when((e == 0) & (f == 0))
    def _():
        out_ref[...] = jnp.zeros_like(out_ref)

    wg_bf[...] = wg_ref[0].astype(jnp.bfloat16)
    wu_bf[...] = wu_ref[0].astype(jnp.bfloat16)
    wd_bf[...] = wd_ref[0].astype(jnp.bfloat16)

    start = st_ref[e]
    end = en_ref[e]
    first = ft_ref[e]

    def chunk(c, carry):
        base = (first + c) * TM
        x = x_ref[pl.ds(base, TM), :]
        gate = jnp.dot(x, wg_bf[...], preferred_element_type=jnp.float32)
        up = jnp.dot(x, wu_bf[...], preferred_element_type=jnp.float32)
        fused = gate * jax.lax.logistic(gate) * up
        fused = fused * w_ref[pl.ds(base, TM), :]
        part = jnp.dot(fused.astype(jnp.bfloat16), wd_bf[...],
                       preferred_element_type=jnp.float32)
        row = base + jax.lax.broadcasted_iota(jnp.int32, (TM, 1), 0)
        mask = (row >= start) & (row < end)
        out_ref[pl.ds(base, TM), :] += jnp.where(mask, part, 0.0)
        return carry

    jax.lax.fori_loop(0, nt_ref[e], chunk, 0)


@functools.cache
def _make_combine(T, D):
    info = plsc.get_sparse_core_info()
    NW = info.num_cores * info.num_subcores
    CT = T // NW
    CH = 32
    NCH = CT // CH
    L = info.num_lanes
    mesh = plsc.VectorSubcoreMesh(core_axis_name="c", subcore_axis_name="s")

    @functools.partial(
        pl.kernel, mesh=mesh,
        out_type=jax.ShapeDtypeStruct((T, D), jnp.float32),
        scratch_types=[
            pltpu.VMEM((CH,), jnp.int32),
            pltpu.VMEM((CH,), jnp.int32),
            pltpu.VMEM((CH, D), jnp.float32),
            pltpu.VMEM((CH, D), jnp.float32),
            pltpu.VMEM((CH, D), jnp.float32),
            pltpu.SemaphoreType.DMA,
            pltpu.SemaphoreType.DMA,
        ])
    def combine(y_hbm, pe_hbm, po_hbm, out_hbm, i0, i1, b0, b1, ob, s0, s1):
        wid = lax.axis_index("s") * info.num_cores + lax.axis_index("c")
        base = wid * CT
        for c in range(NCH):
            tb = base + c * CH
            pltpu.sync_copy(pe_hbm.at[pl.ds(tb, CH)], i0)
            pltpu.sync_copy(po_hbm.at[pl.ds(tb, CH)], i1)
            cp0 = pltpu.async_copy(y_hbm.at[i0], b0, s0)
            cp1 = pltpu.async_copy(y_hbm.at[i1], b1, s1)
            cp0.wait()
            cp1.wait()

            def col(cc, carry):
                off = cc * L
                for j in range(CH):
                    ob[j, pl.ds(off, L)] = (b0[j, pl.ds(off, L)]
                                            + b1[j, pl.ds(off, L)])
                return carry

            lax.fori_loop(0, D // L, col, 0)
            pltpu.sync_copy(ob, out_hbm.at[pl.ds(tb, CH)])

    return combine


@functools.partial(jax.jit, static_argnums=())
def kernel(x_TD, router_weights_TX, selected_experts_TX,
           kernel_gating, kernel_up_proj, kernel_down_proj):
    T, D = x_TD.shape
    K = router_weights_TX.shape[1]
    E, _, F = kernel_gating.shape
    M = T * K
    m_tiles = M // TM
    NF = F // TF

    flat = selected_experts_TX.reshape(-1)
    oh = (flat[:, None] == jnp.arange(E)[None, :]).astype(jnp.int32)
    csum = jnp.cumsum(oh, axis=0)
    sizes = csum[-1]
    ends = jnp.cumsum(sizes)
    starts = ends - sizes
    rank = jnp.sum(oh * csum, axis=1) - 1
    pos = jnp.sum(oh * starts[None, :], axis=1) + rank

    slot_iota = jnp.arange(M, dtype=jnp.int32)
    tok_sorted = jnp.zeros((M,), jnp.int32).at[pos].set(slot_iota // K)
    x_sorted = jnp.take(x_TD.astype(jnp.bfloat16), tok_sorted, axis=0)
    w_sorted = jnp.zeros((M,), jnp.float32).at[pos].set(
        router_weights_TX.reshape(-1))[:, None]

    nonempty = sizes > 0
    first_tile = jnp.where(nonempty, starts // TM, 0).astype(jnp.int32)
    last_tile = jnp.where(nonempty, (ends - 1) // TM, -1)
    ntiles = jnp.maximum(last_tile - first_tile + 1, 0).astype(jnp.int32)

    grid_spec = pltpu.PrefetchScalarGridSpec(
        num_scalar_prefetch=4,
        grid=(E, NF),
        in_specs=[
            pl.BlockSpec((M, D), lambda e, f, ft, nt, st, en: (0, 0)),
            pl.BlockSpec((M, 1), lambda e, f, ft, nt, st, en: (0, 0)),
            pl.BlockSpec((1, D, TF), lambda e, f, ft, nt, st, en: (e, 0, f)),
            pl.BlockSpec((1, D, TF), lambda e, f, ft, nt, st, en: (e, 0, f)),
            pl.BlockSpec((1, TF, D), lambda e, f, ft, nt, st, en: (e, f, 0)),
        ],
        out_specs=pl.BlockSpec((M, D), lambda e, f, ft, nt, st, en: (0, 0)),
        scratch_shapes=[
            pltpu.VMEM((D, TF), jnp.bfloat16),
            pltpu.VMEM((D, TF), jnp.bfloat16),
            pltpu.VMEM((TF, D), jnp.bfloat16),
        ],
    )
    y_sorted = pl.pallas_call(
        _fused_moe_body,
        grid_spec=grid_spec,
        out_shape=jax.ShapeDtypeStruct((M, D), jnp.float32),
        compiler_params=pltpu.CompilerParams(vmem_limit_bytes=62 * 1024 * 1024),
    )(first_tile, ntiles, starts.astype(jnp.int32), ends.astype(jnp.int32),
      x_sorted, w_sorted, kernel_gating, kernel_up_proj, kernel_down_proj)

    pos_TK = pos.reshape(T, K).astype(jnp.int32)
    out_TD = _make_combine(T, D)(y_sorted, pos_TK[:, 0], pos_TK[:, 1])
    return out_TD.astype(jnp.float32)
